# Initial kernel scaffold; baseline (speedup 1.0000x reference)
import jax
import jax.numpy as jnp
from jax.experimental import pallas as pl


def kernel(x, edge_index, edge_weight, encoder_type, W1, b1, W2, b2, W3, b3):
    raise NotImplementedError("write your pallas kernel here")


# speedup vs baseline: 2.5016x; 2.5016x over previous
import functools

import jax
import jax.numpy as jnp
from jax import lax
from jax.experimental import pallas as pl
from jax.experimental.pallas import tpu as pltpu
from jax.experimental.pallas import tpu_sc as plsc

N = 10000
N_PAD = 10240
E = 320000
NS = 16
NC = 2
E_PAD = NS * 20480
E_T = E_PAD // NS
BM = 512


def _mm_body(x_ref, w_ref, o_ref):
    o_ref[...] = jnp.dot(x_ref[...], w_ref[...],
                         preferred_element_type=jnp.float32,
                         precision=lax.Precision.HIGHEST)


def _tc_matmul(x, w):
    m, k = x.shape
    _, n = w.shape
    return pl.pallas_call(
        _mm_body,
        grid=(m // BM,),
        in_specs=[pl.BlockSpec((BM, k), lambda i: (i, 0)),
                  pl.BlockSpec((k, n), lambda i: (0, 0))],
        out_specs=pl.BlockSpec((BM, n), lambda i: (i, 0)),
        out_shape=jax.ShapeDtypeStruct((m, n), jnp.float32),
    )(x, w)


def _brmm_body(h_ref, b_ref, w_ref, o_ref):
    h = jnp.maximum(h_ref[...] + b_ref[...], 0.0)
    o_ref[...] = jnp.dot(h, w_ref[...],
                         preferred_element_type=jnp.float32,
                         precision=lax.Precision.HIGHEST)


def _tc_bias_relu_matmul(h, b, w):
    m, k = h.shape
    _, n = w.shape
    return pl.pallas_call(
        _brmm_body,
        grid=(m // BM,),
        in_specs=[pl.BlockSpec((BM, k), lambda i: (i, 0)),
                  pl.BlockSpec((1, k), lambda i: (0, 0)),
                  pl.BlockSpec((k, n), lambda i: (0, 0))],
        out_specs=pl.BlockSpec((BM, n), lambda i: (i, 0)),
        out_shape=jax.ShapeDtypeStruct((m, n), jnp.float32),
    )(h, b.reshape(1, k), w)


def _head_body(h_ref, b2_ref, w3_ref, b3_ref, ls_ref, pr_ref):
    hp = h_ref[0] + h_ref[1]
    h2 = hp[:, :64] + b2_ref[...]
    mx = jnp.max(h2, axis=1, keepdims=True)
    ex = jnp.exp(h2 - mx)
    ls_ref[...] = (h2 - mx) - jnp.log(jnp.sum(ex, axis=1, keepdims=True))
    pr = jnp.dot(jnp.maximum(h2, 0.0), w3_ref[...],
                 preferred_element_type=jnp.float32,
                 precision=lax.Precision.HIGHEST)
    pr_ref[...] = pr + b3_ref[...]


def _tc_head(h2p, b2, w3, b3):
    _, m, _ = h2p.shape
    k, n = w3.shape
    return pl.pallas_call(
        _head_body,
        grid=(m // BM,),
        in_specs=[pl.BlockSpec((NC, BM, 128), lambda i: (0, i, 0)),
                  pl.BlockSpec((1, k), lambda i: (0, 0)),
                  pl.BlockSpec((k, n), lambda i: (0, 0)),
                  pl.BlockSpec((1, n), lambda i: (0, 0))],
        out_specs=[pl.BlockSpec((BM, k), lambda i: (i, 0)),
                   pl.BlockSpec((BM, n), lambda i: (i, 0))],
        out_shape=[jax.ShapeDtypeStruct((m, k), jnp.float32),
                   jax.ShapeDtypeStruct((m, n), jnp.float32)],
    )(h2p, b2.reshape(1, k), w3, b3.reshape(1, n))


SUP = 1024
N_SUP = E_T // SUP


def _make_spmm(col_split):
    rpt = N_PAD // NS
    iters = N_SUP if col_split else N_SUP // 2
    table_rows = NC * N_PAD if col_split else N_PAD
    mesh = plsc.VectorSubcoreMesh(core_axis_name="c", subcore_axis_name="s")

    @functools.partial(
        pl.kernel,
        out_type=jax.ShapeDtypeStruct((NC, N_PAD, 128), jnp.float32),
        mesh=mesh,
        scratch_types=[
            pltpu.VMEM((8, 128), jnp.int32),
            pltpu.VMEM((8, 128), jnp.int32),
            pltpu.VMEM((8, 128), jnp.float32),
            pltpu.VMEM((128, 128), jnp.float32),
            pltpu.VMEM_SHARED((N_PAD, 128), jnp.float32),
            pltpu.SemaphoreType.DMA,
        ],
    )
    def spmm(src_hbm, dst_hbm, w_hbm, table_hbm, zeros_hbm, out_hbm,
             srcb, dstb, wb, rows, acc, sem):
        cid = lax.axis_index("c")
        sid = lax.axis_index("s")
        r0 = sid * rpt
        pltpu.sync_copy(zeros_hbm.at[pl.ds(r0, rpt)], acc.at[pl.ds(r0, rpt)])
        plsc.subcore_barrier()

        if col_split:
            base = sid * N_SUP
        else:
            base = (cid * NS + sid) * iters

        def chunk_body(k, carry):
            blk = base + k
            pltpu.sync_copy(src_hbm.at[blk], srcb)
            pltpu.sync_copy(dst_hbm.at[blk], dstb)
            pltpu.sync_copy(w_hbm.at[blk], wb)
            if col_split:
                off = cid * N_PAD
                for j in range(8):
                    for v in range(8):
                        sl = pl.ds(v * 16, 16)
                        srcb[j, sl] = srcb[j, sl] + off
            for j in range(8):
                pltpu.async_copy(table_hbm.at[srcb.at[j]], rows, sem).wait()

                def scale(g, c2):
                    w16 = wb[j, pl.ds(g * 16, 16)]
                    e0 = g * 16
                    for l in range(16):
                        wsc = w16[l]
                        for v in range(8):
                            sl = pl.ds(v * 16, 16)
                            rows[e0 + l, sl] = rows[e0 + l, sl] * wsc
                    return c2
                lax.fori_loop(0, 8, scale, 0)
                pltpu.sync_copy(rows, acc.at[dstb.at[j]], add=True)
            return carry

        lax.fori_loop(0, iters, chunk_body, 0)
        plsc.subcore_barrier()
        pltpu.sync_copy(acc.at[pl.ds(r0, rpt)],
                        out_hbm.at[cid, pl.ds(r0, rpt), :])

    return spmm


_spmm_cols = _make_spmm(True)
_spmm_edges = _make_spmm(False)


def kernel(x, edge_index, edge_weight, encoder_type, W1, b1, W2, b2, W3, b3):
    xp = jnp.pad(x, ((0, N_PAD - N), (0, 0)))
    src = jnp.pad(edge_index[0], (0, E_PAD - E)).reshape(E_PAD // SUP, 8, 128)
    dst = jnp.pad(edge_index[1], (0, E_PAD - E)).reshape(E_PAD // SUP, 8, 128)
    ew = jnp.pad(edge_weight, (0, E_PAD - E)).reshape(E_PAD // SUP, 8, 128)

    zeros = jnp.zeros((N_PAD, 128), jnp.float32)

    xw = _tc_matmul(xp, W1)
    t1 = xw.reshape(N_PAD, 2, 128).transpose(1, 0, 2).reshape(2 * N_PAD, 128)
    hpre = _spmm_cols(src, dst, ew, t1, zeros)
    h = hpre.transpose(1, 0, 2).reshape(N_PAD, 256)

    hw = _tc_bias_relu_matmul(h, b1, W2)
    t2 = jnp.pad(hw, ((0, 0), (0, 64)))
    h2p = _spmm_edges(src, dst, ew, t2, zeros)

    ls, pr = _tc_head(h2p, b2, W3, b3)
    return ls[:N], pr[:N]


# speedup vs baseline: 3.0161x; 1.2057x over previous
import functools

import jax
import jax.numpy as jnp
from jax import lax
from jax.experimental import pallas as pl
from jax.experimental.pallas import tpu as pltpu
from jax.experimental.pallas import tpu_sc as plsc

N = 10000
N_PAD = 10240
E = 320000
NS = 16
NC = 2
E_PAD = NS * 20480
E_T = E_PAD // NS
BM = 512


def _mm_body(x_ref, w_ref, o_ref):
    o_ref[...] = jnp.dot(x_ref[...], w_ref[...],
                         preferred_element_type=jnp.float32,
                         precision=lax.Precision.HIGHEST)


def _tc_matmul(x, w):
    m, k = x.shape
    _, n = w.shape
    return pl.pallas_call(
        _mm_body,
        grid=(m // BM,),
        in_specs=[pl.BlockSpec((BM, k), lambda i: (i, 0)),
                  pl.BlockSpec((k, n), lambda i: (0, 0))],
        out_specs=pl.BlockSpec((BM, n), lambda i: (i, 0)),
        out_shape=jax.ShapeDtypeStruct((m, n), jnp.float32),
    )(x, w)


def _brmm_body(h_ref, b_ref, w_ref, o_ref):
    h = jnp.maximum(h_ref[...] + b_ref[...], 0.0)
    o_ref[...] = jnp.dot(h, w_ref[...],
                         preferred_element_type=jnp.float32,
                         precision=lax.Precision.HIGHEST)


def _tc_bias_relu_matmul(h, b, w):
    m, k = h.shape
    _, n = w.shape
    return pl.pallas_call(
        _brmm_body,
        grid=(m // BM,),
        in_specs=[pl.BlockSpec((BM, k), lambda i: (i, 0)),
                  pl.BlockSpec((1, k), lambda i: (0, 0)),
                  pl.BlockSpec((k, n), lambda i: (0, 0))],
        out_specs=pl.BlockSpec((BM, n), lambda i: (i, 0)),
        out_shape=jax.ShapeDtypeStruct((m, n), jnp.float32),
    )(h, b.reshape(1, k), w)


def _head_body(h_ref, b2_ref, w3_ref, b3_ref, ls_ref, pr_ref):
    hp = h_ref[0] + h_ref[1]
    h2 = hp[:, :64] + b2_ref[...]
    mx = jnp.max(h2, axis=1, keepdims=True)
    ex = jnp.exp(h2 - mx)
    ls_ref[...] = (h2 - mx) - jnp.log(jnp.sum(ex, axis=1, keepdims=True))
    pr = jnp.dot(jnp.maximum(h2, 0.0), w3_ref[...],
                 preferred_element_type=jnp.float32,
                 precision=lax.Precision.HIGHEST)
    pr_ref[...] = pr + b3_ref[...]


def _tc_head(h2p, b2, w3, b3):
    _, m, _ = h2p.shape
    k, n = w3.shape
    return pl.pallas_call(
        _head_body,
        grid=(m // BM,),
        in_specs=[pl.BlockSpec((NC, BM, 128), lambda i: (0, i, 0)),
                  pl.BlockSpec((1, k), lambda i: (0, 0)),
                  pl.BlockSpec((k, n), lambda i: (0, 0)),
                  pl.BlockSpec((1, n), lambda i: (0, 0))],
        out_specs=[pl.BlockSpec((BM, k), lambda i: (i, 0)),
                   pl.BlockSpec((BM, n), lambda i: (i, 0))],
        out_shape=[jax.ShapeDtypeStruct((m, k), jnp.float32),
                   jax.ShapeDtypeStruct((m, n), jnp.float32)],
    )(h2p, b2.reshape(1, k), w3, b3.reshape(1, n))


SUP = 1024
N_SUP = E_T // SUP


def _make_spmm(col_split):
    rpt = N_PAD // NS
    iters = N_SUP if col_split else N_SUP // 2
    G = iters * 8
    mesh = plsc.VectorSubcoreMesh(core_axis_name="c", subcore_axis_name="s")

    @functools.partial(
        pl.kernel,
        out_type=jax.ShapeDtypeStruct((NC, N_PAD, 128), jnp.float32),
        mesh=mesh,
        scratch_types=[
            pltpu.VMEM((16, 128), jnp.int32),
            pltpu.VMEM((16, 128), jnp.int32),
            pltpu.VMEM((16, 128), jnp.float32),
            pltpu.VMEM((2 * 128, 128), jnp.float32),
            pltpu.VMEM_SHARED((N_PAD, 128), jnp.float32),
            pltpu.SemaphoreType.DMA,
            pltpu.SemaphoreType.DMA,
            pltpu.SemaphoreType.DMA,
            pltpu.SemaphoreType.DMA,
            pltpu.SemaphoreType.DMA,
        ],
    )
    def spmm(src_hbm, dst_hbm, w_hbm, table_hbm, zeros_hbm, out_hbm,
             srcb, dstb, wb, rows, acc, gsem0, gsem1, ssem0, ssem1, isem):
        cid = lax.axis_index("c")
        sid = lax.axis_index("s")
        r0 = sid * rpt
        if col_split:
            base = sid * iters
            off = cid * N_PAD
        else:
            base = (cid * NS + sid) * iters
            off = None

        def add_off(buf_sel):
            if col_split:
                for j in range(8):
                    for v in range(8):
                        sl = pl.ds(v * 16, 16)
                        srcb[buf_sel * 8 + j, sl] = srcb[buf_sel * 8 + j, sl] + off

        gsems = (gsem0, gsem1)
        ssems = (ssem0, ssem1)

        def gather_cp(gg, rb):
            kg = gg // 8
            row = (kg % 2) * 8 + (gg % 8)
            return pltpu.make_async_copy(
                table_hbm.at[srcb.at[row]], rows.at[pl.ds(rb * 128, 128)],
                gsems[rb])

        def scatter_cp(rb, drow):
            return pltpu.make_async_copy(
                rows.at[pl.ds(rb * 128, 128)], acc.at[dstb.at[drow]],
                ssems[rb])

        def idx_cp(blk, buf_sel, which):
            hbm = (src_hbm, dst_hbm, w_hbm)[which]
            buf = (srcb, dstb, wb)[which]
            return pltpu.make_async_copy(
                hbm.at[blk], buf.at[pl.ds(buf_sel * 8, 8)], isem)

        for which in range(3):
            idx_cp(base, 0, which).start()
        pltpu.sync_copy(zeros_hbm.at[pl.ds(r0, rpt)], acc.at[pl.ds(r0, rpt)])
        for which in range(3):
            idx_cp(base, 0, which).wait()
        add_off(0)
        plsc.subcore_barrier()
        gather_cp(0, 0).start()
        T = G // 2

        def body(t, carry):
            for b in (0, 1):
                g = 2 * t + b
                k = g // 8
                j = g % 8
                nb = 1 - b

                if b == 0:
                    @pl.when(j == 2)
                    def _prefetch():
                        kn = jnp.minimum(k + 1, iters - 1)
                        for which in range(3):
                            idx_cp(base + kn, (k + 1) % 2, which).start()

                    @pl.when(t >= 1)
                    def _buf_free():
                        scatter_cp(nb, 0).wait()

                    gather_cp(g + 1, nb).start()
                else:
                    @pl.when(j == 5)
                    def _landed():
                        for which in range(3):
                            idx_cp(base, 0, which).wait()
                        add_off((k + 1) % 2)

                    scatter_cp(nb, 0).wait()

                    @pl.when(t < T - 1)
                    def _next_gather():
                        gather_cp(g + 1, nb).start()

                gather_cp(g, b).wait()
                rb = b * 128
                kb = k % 2

                def scale(grp, c2):
                    w16 = wb[kb * 8 + j, pl.ds(grp * 16, 16)]
                    e0 = rb + grp * 16
                    for l in range(16):
                        wsc = w16[l]
                        for v in range(8):
                            sl = pl.ds(v * 16, 16)
                            rows[e0 + l, sl] = rows[e0 + l, sl] * wsc
                    return c2
                lax.fori_loop(0, 8, scale, 0)

                scatter_cp(b, kb * 8 + j).start(add=True)
            return carry

        lax.fori_loop(0, T, body, 0)

        scatter_cp(1, 0).wait()
        plsc.subcore_barrier()
        pltpu.sync_copy(acc.at[pl.ds(r0, rpt)],
                        out_hbm.at[cid, pl.ds(r0, rpt), :])

    return spmm


_spmm_cols = _make_spmm(True)
_spmm_edges = _make_spmm(False)


def kernel(x, edge_index, edge_weight, encoder_type, W1, b1, W2, b2, W3, b3):
    xp = jnp.pad(x, ((0, N_PAD - N), (0, 0)))
    src = jnp.pad(edge_index[0], (0, E_PAD - E)).reshape(E_PAD // SUP, 8, 128)
    dst = jnp.pad(edge_index[1], (0, E_PAD - E)).reshape(E_PAD // SUP, 8, 128)
    ew = jnp.pad(edge_weight, (0, E_PAD - E)).reshape(E_PAD // SUP, 8, 128)

    zeros = jnp.zeros((N_PAD, 128), jnp.float32)

    xw = _tc_matmul(xp, W1)
    t1 = xw.reshape(N_PAD, 2, 128).transpose(1, 0, 2).reshape(2 * N_PAD, 128)
    hpre = _spmm_cols(src, dst, ew, t1, zeros)
    h = hpre.transpose(1, 0, 2).reshape(N_PAD, 256)

    hw = _tc_bias_relu_matmul(h, b1, W2)
    t2 = jnp.pad(hw, ((0, 0), (0, 64)))
    h2p = _spmm_edges(src, dst, ew, t2, zeros)

    ls, pr = _tc_head(h2p, b2, W3, b3)
    return ls[:N], pr[:N]
